# baseline (speedup 1.0000x reference)
import functools

import jax
import jax.numpy as jnp
from jax import lax
from jax.experimental import pallas as pl
from jax.experimental.pallas import tpu as pltpu
from jax.experimental.pallas import tpu_sc as plsc

N = 10000
E = 160000
W = 128
RB = 400
NRB = N // RB
RPT = 632
RPT_LAST = N - 15 * RPT

_f32 = jnp.float32


def _dot(a, b):
    return jnp.dot(a, b, preferred_element_type=_f32,
                   precision=jax.lax.Precision.DEFAULT)


def _sc_mesh():
    return plsc.VectorSubcoreMesh(core_axis_name="c", subcore_axis_name="s")


def _zero_acc(sid, zeros_hbm, acc):
    @pl.when(sid < 15)
    def _():
        row0 = pl.multiple_of(sid * RPT, 8)
        pltpu.sync_copy(zeros_hbm, acc.at[pl.ds(row0, RPT)])

    @pl.when(sid == 15)
    def _():
        pltpu.sync_copy(zeros_hbm.at[pl.ds(0, RPT_LAST)],
                        acc.at[pl.ds(15 * RPT, RPT_LAST)])


def _copy_acc_out(sid, acc, out_hbm, out_off):
    @pl.when(sid < 15)
    def _():
        row0 = pl.multiple_of(sid * RPT, 8)
        pltpu.sync_copy(acc.at[pl.ds(row0, RPT)],
                        out_hbm.at[pl.ds(pl.multiple_of(out_off + row0, 8), RPT)])

    @pl.when(sid == 15)
    def _():
        pltpu.sync_copy(
            acc.at[pl.ds(15 * RPT, RPT_LAST)],
            out_hbm.at[pl.ds(pl.multiple_of(out_off + 15 * RPT, 8), RPT_LAST)])


@functools.lru_cache(maxsize=None)
def _build_degree():
    B = 40
    EPT = E // 32
    NB = EPT // B

    @functools.partial(
        pl.kernel,
        out_type=jax.ShapeDtypeStruct((2 * N, W), _f32),
        mesh=_sc_mesh(),
        scratch_types=[
            pltpu.VMEM((B,), jnp.int32),
            pltpu.VMEM((B, W), _f32),
            pltpu.VMEM_SHARED((N, W), _f32),
        ],
    )
    def deg_kernel(dst_hbm, ones_hbm, zeros_hbm, out_hbm, didx, ones_v, acc):
        cid = lax.axis_index("c")
        sid = lax.axis_index("s")
        e0 = pl.multiple_of((cid * 16 + sid) * EPT, 8)
        pltpu.sync_copy(ones_hbm, ones_v)
        _zero_acc(sid, zeros_hbm, acc)
        plsc.subcore_barrier()

        def batch(b, carry):
            off = pl.multiple_of(e0 + b * B, 8)
            pltpu.sync_copy(dst_hbm.at[pl.ds(off, B)], didx)
            pltpu.sync_copy(ones_v, acc.at[didx], add=True)
            return carry

        lax.fori_loop(0, NB, batch, 0)
        plsc.subcore_barrier()
        _copy_acc_out(sid, acc, out_hbm, cid * N)

    return deg_kernel


@functools.lru_cache(maxsize=None)
def _build_aggregate(n_chunks: int, edge_split: bool):
    K = 5
    if edge_split:
        assert n_chunks == 1
        B = 40
        EPT = E // 32
        c_half = 1
        out_rows = 2 * N
    else:
        assert n_chunks % 2 == 0
        B = 40
        EPT = E // 16
        c_half = n_chunks // 2
        out_rows = n_chunks * N
    NB = EPT // B
    NSB = NB // K
    assert NSB * K == NB
    loop_pairs = (NSB - 2) // 2
    statics = list(range(2 * loop_pairs, NSB))

    @functools.partial(
        pl.kernel,
        out_type=jax.ShapeDtypeStruct((out_rows, W), _f32),
        mesh=_sc_mesh(),
        scratch_types=[
            pltpu.VMEM((K, B), jnp.int32),
            pltpu.VMEM((K, B), jnp.int32),
            pltpu.VMEM((K, B), jnp.int32),
            pltpu.VMEM((K, B), jnp.int32),
            pltpu.VMEM((K, B, W), _f32),
            pltpu.SemaphoreType.DMA,
        ] + [pltpu.SemaphoreType.DMA] * K + [
            pltpu.VMEM_SHARED((N, W), _f32),
        ],
    )
    def agg_kernel(srcx_hbm, dst_hbm, tbl_hbm, zeros_hbm, out_hbm,
                   sidxA, didxA, sidxB, didxB, rows, isem,
                   *sems_and_acc):
        gsem = sems_and_acc[:K]
        acc = sems_and_acc[K]
        cid = lax.axis_index("c")
        sid = lax.axis_index("s")
        if edge_split:
            e0 = pl.multiple_of((cid * 16 + sid) * EPT, 8)
        else:
            e0 = pl.multiple_of(sid * EPT, 8)

        for j in range(c_half):
            if edge_split:
                chunk = 0
                out_off = cid * N
            else:
                chunk = cid * c_half + j
                out_off = (cid * c_half + j) * N
            _zero_acc(sid, zeros_hbm, acc)
            plsc.subcore_barrier()

            def fire_idx(m, ss, dd):
                copies = []
                for t in range(K):
                    off = pl.multiple_of(e0 + m * (K * B) + t * B, 8)
                    copies.append(pltpu.async_copy(
                        srcx_hbm.at[pl.ds(chunk * E + off, B)], ss.at[t],
                        isem))
                    copies.append(pltpu.async_copy(
                        dst_hbm.at[pl.ds(off, B)], dd.at[t], isem))
                return copies

            def process(m, cs, cd, ns, nd, fire_next):
                icopies = fire_idx(m + 1, ns, nd) if fire_next else []
                for t in range(K):
                    pltpu.make_async_copy(
                        tbl_hbm.at[cs.at[t]], rows.at[t], gsem[t]).wait()
                    pltpu.sync_copy(rows.at[t], acc.at[cd.at[t]], add=True)
                    if fire_next:
                        if t == 0:
                            for c in icopies:
                                c.wait()
                        pltpu.async_copy(
                            tbl_hbm.at[ns.at[t]], rows.at[t], gsem[t])

            for c in fire_idx(0, sidxA, didxA):
                c.wait()
            for t in range(K):
                pltpu.async_copy(tbl_hbm.at[sidxA.at[t]], rows.at[t], gsem[t])

            def pair(q, carry):
                m0 = q * 2
                process(m0, sidxA, didxA, sidxB, didxB, True)
                process(m0 + 1, sidxB, didxB, sidxA, didxA, True)
                return carry

            lax.fori_loop(0, loop_pairs, pair, 0)
            for m in statics:
                if m % 2 == 0:
                    process(m, sidxA, didxA, sidxB, didxB, m < NSB - 1)
                else:
                    process(m, sidxB, didxB, sidxA, didxA, m < NSB - 1)

            plsc.subcore_barrier()
            _copy_acc_out(sid, acc, out_hbm, out_off)
            plsc.subcore_barrier()

    return agg_kernel


def _sc_degree(dst):
    ones = jnp.ones((40, W), _f32)
    zeros = jnp.zeros((RPT, W), _f32)
    return _build_degree()(dst, ones, zeros)


def _sc_aggregate(srcx, dst, table, n_chunks, edge_split):
    zeros = jnp.zeros((RPT, W), _f32)
    return _build_aggregate(n_chunks, edge_split)(srcx, dst, table, zeros)


def _chunked_src(src, n_chunks):
    EB = 16000

    def body(s_ref, o_ref):
        off = jax.lax.broadcasted_iota(jnp.int32, (n_chunks, EB), 0) * N
        o_ref[...] = s_ref[...] + off

    out = pl.pallas_call(
        body,
        grid=(E // EB,),
        in_specs=[pl.BlockSpec((1, EB), lambda i: (0, i))],
        out_specs=pl.BlockSpec((n_chunks, EB), lambda i: (0, i)),
        out_shape=jax.ShapeDtypeStruct((n_chunks, E), jnp.int32),
    )(src.reshape(1, E))
    return out.reshape(-1)


def _ablock(d0_ref, d1_ref):
    return jax.lax.rsqrt(d0_ref[:, 0:1] + d1_ref[:, 0:1] + 1.0)


_DSPEC0 = pl.BlockSpec((RB, W), lambda i, *_: (i, 0))
_DSPEC1 = pl.BlockSpec((RB, W), lambda i, *_: (i + NRB, 0))


def _k0(x, dparts):
    def body(x_ref, d0, d1, o_ref):
        o_ref[...] = _ablock(d0, d1) * x_ref[...]

    return pl.pallas_call(
        body,
        grid=(NRB,),
        in_specs=[pl.BlockSpec((RB, W), lambda i: (i, 0)), _DSPEC0, _DSPEC1],
        out_specs=pl.BlockSpec((RB, W), lambda i: (i, 0)),
        out_shape=jax.ShapeDtypeStruct((N, W), _f32),
    )(x, dparts, dparts)


def _k1(parts, g0, dparts, w1, b1):
    def body(p0, p1, g0_ref, d0, d1, w_ref, b_ref, o_ref):
        a = _ablock(d0, d1)
        m = a * (p0[...] + p1[...] + g0_ref[...])
        h = a * jnp.tanh(_dot(m, w_ref[...]) + b_ref[...])
        for j in range(16):
            o_ref[j] = h[:, j * W:(j + 1) * W]

    return pl.pallas_call(
        body,
        grid=(NRB,),
        in_specs=[
            pl.BlockSpec((RB, W), lambda i: (i, 0)),
            pl.BlockSpec((RB, W), lambda i: (i + NRB, 0)),
            pl.BlockSpec((RB, W), lambda i: (i, 0)),
            _DSPEC0, _DSPEC1,
            pl.BlockSpec((W, 2048), lambda i: (0, 0)),
            pl.BlockSpec((1, 2048), lambda i: (0, 0)),
        ],
        out_specs=pl.BlockSpec((16, RB, W), lambda i: (0, i, 0)),
        out_shape=jax.ShapeDtypeStruct((16, N, W), _f32),
    )(parts, parts, g0, dparts, dparts, w1, b1)


def _k2a(agg1, g1, dparts, w2, b2):
    def body(agg_ref, g_ref, d0, d1, w_ref, b_ref, o_ref):
        a = _ablock(d0, d1)
        acc = jnp.zeros((RB, 2048), _f32)
        for k in range(16):
            m = a * (agg_ref[k] + g_ref[k])
            acc = acc + _dot(m, w_ref[k * W:(k + 1) * W, :])
        o_ref[...] = jnp.tanh(acc + b_ref[...])

    return pl.pallas_call(
        body,
        grid=(NRB,),
        in_specs=[
            pl.BlockSpec((16, RB, W), lambda i: (0, i, 0)),
            pl.BlockSpec((16, RB, W), lambda i: (0, i, 0)),
            _DSPEC0, _DSPEC1,
            pl.BlockSpec((2048, 2048), lambda i: (0, 0)),
            pl.BlockSpec((1, 2048), lambda i: (0, 0)),
        ],
        out_specs=pl.BlockSpec((RB, 2048), lambda i: (i, 0)),
        out_shape=jax.ShapeDtypeStruct((N, 2048), _f32),
    )(agg1, g1, dparts, dparts, w2, b2)


def _k2b(h2, dparts, w3):
    def body(h_ref, d0, d1, w_ref, o_ref):
        a = _ablock(d0, d1)
        v = a * _dot(h_ref[...], w_ref[...])
        for j in range(8):
            o_ref[j] = v[:, j * W:(j + 1) * W]

    return pl.pallas_call(
        body,
        grid=(NRB,),
        in_specs=[
            pl.BlockSpec((RB, 2048), lambda i: (i, 0)),
            _DSPEC0, _DSPEC1,
            pl.BlockSpec((2048, 1024), lambda i: (0, 0)),
        ],
        out_specs=pl.BlockSpec((8, RB, W), lambda i: (0, i, 0)),
        out_shape=jax.ShapeDtypeStruct((8, N, W), _f32),
    )(h2, dparts, dparts, w3)


def _k3(agg2, g2, dparts, b3, w4, b4):
    def body(agg_ref, g_ref, d0, d1, b3_ref, w_ref, b4_ref, o_ref):
        a = _ablock(d0, d1)
        acc = jnp.zeros((RB, 768), _f32)
        for k in range(8):
            t = jnp.tanh(a * (agg_ref[k] + g_ref[k])
                         + b3_ref[:, k * W:(k + 1) * W])
            acc = acc + _dot(t, w_ref[k * W:(k + 1) * W, :])
        o_ref[...] = acc + b4_ref[...]

    return pl.pallas_call(
        body,
        grid=(NRB,),
        in_specs=[
            pl.BlockSpec((8, RB, W), lambda i: (0, i, 0)),
            pl.BlockSpec((8, RB, W), lambda i: (0, i, 0)),
            _DSPEC0, _DSPEC1,
            pl.BlockSpec((1, 1024), lambda i: (0, 0)),
            pl.BlockSpec((1024, 768), lambda i: (0, 0)),
            pl.BlockSpec((1, 768), lambda i: (0, 0)),
        ],
        out_specs=pl.BlockSpec((RB, 768), lambda i: (i, 0)),
        out_shape=jax.ShapeDtypeStruct((N, 768), _f32),
    )(agg2, g2, dparts, dparts, b3, w4, b4)


def kernel(x, edge_index, W1, b1, W2, b2, W3, b3, W4, b4):
    ei = edge_index.astype(jnp.int32)
    src, dst = ei[0], ei[1]

    srcx16 = _chunked_src(src, 16)

    dparts = _sc_degree(dst)
    g0 = _k0(x, dparts)
    p = _sc_aggregate(src, dst, g0, 1, True)
    g1 = _k1(p, g0, dparts, W1, b1.reshape(1, -1))
    agg1 = _sc_aggregate(srcx16, dst, g1.reshape(16 * N, W), 16, False)
    h2 = _k2a(agg1.reshape(16, N, W), g1, dparts, W2, b2.reshape(1, -1))
    g2 = _k2b(h2, dparts, W3)
    agg2 = _sc_aggregate(srcx16[:8 * E], dst, g2.reshape(8 * N, W), 8, False)
    out = _k3(agg2.reshape(8, N, W), g2, dparts,
              b3.reshape(1, -1), W4, b4.reshape(1, -1))
    return out


# speedup vs baseline: 1.0283x; 1.0283x over previous
import functools

import jax
import jax.numpy as jnp
from jax import lax
from jax.experimental import pallas as pl
from jax.experimental.pallas import tpu as pltpu
from jax.experimental.pallas import tpu_sc as plsc

N = 10000
E = 160000
W = 128
RB = 400
NRB = N // RB
RPT = 632
RPT_LAST = N - 15 * RPT

_f32 = jnp.float32


def _dot(a, b):
    return jnp.dot(a, b, preferred_element_type=_f32,
                   precision=jax.lax.Precision.DEFAULT)


def _sc_mesh():
    return plsc.VectorSubcoreMesh(core_axis_name="c", subcore_axis_name="s")


def _zero_acc(sid, zeros_hbm, acc):
    @pl.when(sid < 15)
    def _():
        row0 = pl.multiple_of(sid * RPT, 8)
        pltpu.sync_copy(zeros_hbm, acc.at[pl.ds(row0, RPT)])

    @pl.when(sid == 15)
    def _():
        pltpu.sync_copy(zeros_hbm.at[pl.ds(0, RPT_LAST)],
                        acc.at[pl.ds(15 * RPT, RPT_LAST)])


def _copy_acc_out(sid, acc, out_hbm, out_off):
    @pl.when(sid < 15)
    def _():
        row0 = pl.multiple_of(sid * RPT, 8)
        pltpu.sync_copy(acc.at[pl.ds(row0, RPT)],
                        out_hbm.at[pl.ds(pl.multiple_of(out_off + row0, 8), RPT)])

    @pl.when(sid == 15)
    def _():
        pltpu.sync_copy(
            acc.at[pl.ds(15 * RPT, RPT_LAST)],
            out_hbm.at[pl.ds(pl.multiple_of(out_off + 15 * RPT, 8), RPT_LAST)])


@functools.lru_cache(maxsize=None)
def _build_degree():
    B = 40
    EPT = E // 32
    NB = EPT // B

    @functools.partial(
        pl.kernel,
        out_type=jax.ShapeDtypeStruct((2 * N, W), _f32),
        mesh=_sc_mesh(),
        scratch_types=[
            pltpu.VMEM((NB, B), jnp.int32),
            pltpu.VMEM((B, W), _f32),
            pltpu.SemaphoreType.DMA,
            pltpu.VMEM_SHARED((N, W), _f32),
        ],
    )
    def deg_kernel(dst_hbm, ones_hbm, zeros_hbm, out_hbm, didx2, ones_v,
                   isem, acc):
        cid = lax.axis_index("c")
        sid = lax.axis_index("s")
        e0 = pl.multiple_of((cid * 16 + sid) * EPT, 8)
        pltpu.sync_copy(ones_hbm, ones_v)
        _zero_acc(sid, zeros_hbm, acc)

        def load(b, carry):
            off = pl.multiple_of(e0 + b * B, 8)
            pltpu.async_copy(dst_hbm.at[pl.ds(off, B)], didx2.at[b], isem)
            return carry

        lax.fori_loop(0, NB, load, 0)

        def drain(b, carry):
            off = pl.multiple_of(e0 + b * B, 8)
            pltpu.make_async_copy(dst_hbm.at[pl.ds(off, B)], didx2.at[b],
                                  isem).wait()
            return carry

        lax.fori_loop(0, NB, drain, 0)
        plsc.subcore_barrier()

        def batch(b, carry):
            pltpu.sync_copy(ones_v, acc.at[didx2.at[b]], add=True)
            return carry

        lax.fori_loop(0, NB, batch, 0)
        plsc.subcore_barrier()
        _copy_acc_out(sid, acc, out_hbm, cid * N)

    return deg_kernel


@functools.lru_cache(maxsize=None)
def _build_aggregate(n_chunks: int, edge_split: bool):
    K = 5
    if edge_split:
        assert n_chunks == 1
        B = 40
        EPT = E // 32
        c_half = 1
        out_rows = 2 * N
    else:
        assert n_chunks % 2 == 0
        B = 40
        EPT = E // 16
        c_half = n_chunks // 2
        out_rows = n_chunks * N
    NB = EPT // B
    NSB = NB // K
    assert NSB * K == NB
    loop_pairs = (NSB - 2) // 2
    statics = list(range(2 * loop_pairs, NSB))

    @functools.partial(
        pl.kernel,
        out_type=jax.ShapeDtypeStruct((out_rows, W), _f32),
        mesh=_sc_mesh(),
        scratch_types=[
            pltpu.VMEM((K, B), jnp.int32),
            pltpu.VMEM((K, B), jnp.int32),
            pltpu.VMEM((K, B), jnp.int32),
            pltpu.VMEM((K, B), jnp.int32),
            pltpu.VMEM((K, B, W), _f32),
            pltpu.SemaphoreType.DMA,
        ] + [pltpu.SemaphoreType.DMA] * K + [
            pltpu.VMEM_SHARED((N, W), _f32),
        ],
    )
    def agg_kernel(srcx_hbm, dst_hbm, tbl_hbm, zeros_hbm, out_hbm,
                   sidxA, didxA, sidxB, didxB, rows, isem,
                   *sems_and_acc):
        gsem = sems_and_acc[:K]
        acc = sems_and_acc[K]
        cid = lax.axis_index("c")
        sid = lax.axis_index("s")
        if edge_split:
            e0 = pl.multiple_of((cid * 16 + sid) * EPT, 8)
        else:
            e0 = pl.multiple_of(sid * EPT, 8)

        for j in range(c_half):
            if edge_split:
                chunk = 0
                out_off = cid * N
            else:
                chunk = cid * c_half + j
                out_off = (cid * c_half + j) * N
            _zero_acc(sid, zeros_hbm, acc)
            plsc.subcore_barrier()

            def fire_idx(m, ss, dd):
                copies = []
                for t in range(K):
                    off = pl.multiple_of(e0 + m * (K * B) + t * B, 8)
                    copies.append(pltpu.async_copy(
                        srcx_hbm.at[pl.ds(chunk * E + off, B)], ss.at[t],
                        isem))
                    copies.append(pltpu.async_copy(
                        dst_hbm.at[pl.ds(off, B)], dd.at[t], isem))
                return copies

            def process(m, cs, cd, ns, nd, fire_next):
                icopies = fire_idx(m + 1, ns, nd) if fire_next else []
                for t in range(K):
                    pltpu.make_async_copy(
                        tbl_hbm.at[cs.at[t]], rows.at[t], gsem[t]).wait()
                    pltpu.sync_copy(rows.at[t], acc.at[cd.at[t]], add=True)
                    if fire_next:
                        if t == 0:
                            for c in icopies:
                                c.wait()
                        pltpu.async_copy(
                            tbl_hbm.at[ns.at[t]], rows.at[t], gsem[t])

            for c in fire_idx(0, sidxA, didxA):
                c.wait()
            for t in range(K):
                pltpu.async_copy(tbl_hbm.at[sidxA.at[t]], rows.at[t], gsem[t])

            def pair(q, carry):
                m0 = q * 2
                process(m0, sidxA, didxA, sidxB, didxB, True)
                process(m0 + 1, sidxB, didxB, sidxA, didxA, True)
                return carry

            lax.fori_loop(0, loop_pairs, pair, 0)
            for m in statics:
                if m % 2 == 0:
                    process(m, sidxA, didxA, sidxB, didxB, m < NSB - 1)
                else:
                    process(m, sidxB, didxB, sidxA, didxA, m < NSB - 1)

            plsc.subcore_barrier()
            _copy_acc_out(sid, acc, out_hbm, out_off)
            plsc.subcore_barrier()

    return agg_kernel


def _sc_degree(dst):
    ones = jnp.ones((40, W), _f32)
    zeros = jnp.zeros((RPT, W), _f32)
    return _build_degree()(dst, ones, zeros)


def _sc_aggregate(srcx, dst, table, n_chunks, edge_split):
    zeros = jnp.zeros((RPT, W), _f32)
    return _build_aggregate(n_chunks, edge_split)(srcx, dst, table, zeros)


def _chunked_src(src, n_chunks):
    EB = 16000

    def body(s_ref, o_ref):
        off = jax.lax.broadcasted_iota(jnp.int32, (n_chunks, EB), 0) * N
        o_ref[...] = s_ref[...] + off

    out = pl.pallas_call(
        body,
        grid=(E // EB,),
        in_specs=[pl.BlockSpec((1, EB), lambda i: (0, i))],
        out_specs=pl.BlockSpec((n_chunks, EB), lambda i: (0, i)),
        out_shape=jax.ShapeDtypeStruct((n_chunks, E), jnp.int32),
    )(src.reshape(1, E))
    return out.reshape(-1)


def _ablock(d0_ref, d1_ref):
    return jax.lax.rsqrt(d0_ref[:, 0:1] + d1_ref[:, 0:1] + 1.0)


_DSPEC0 = pl.BlockSpec((RB, W), lambda i, *_: (i, 0))
_DSPEC1 = pl.BlockSpec((RB, W), lambda i, *_: (i + NRB, 0))


def _k0(x, dparts):
    def body(x_ref, d0, d1, o_ref):
        o_ref[...] = _ablock(d0, d1) * x_ref[...]

    return pl.pallas_call(
        body,
        grid=(NRB,),
        in_specs=[pl.BlockSpec((RB, W), lambda i: (i, 0)), _DSPEC0, _DSPEC1],
        out_specs=pl.BlockSpec((RB, W), lambda i: (i, 0)),
        out_shape=jax.ShapeDtypeStruct((N, W), _f32),
    )(x, dparts, dparts)


def _k1(parts, g0, dparts, w1, b1):
    def body(p0, p1, g0_ref, d0, d1, w_ref, b_ref, o_ref):
        a = _ablock(d0, d1)
        m = a * (p0[...] + p1[...] + g0_ref[...])
        h = a * jnp.tanh(_dot(m, w_ref[...]) + b_ref[...])
        for j in range(16):
            o_ref[j] = h[:, j * W:(j + 1) * W]

    return pl.pallas_call(
        body,
        grid=(NRB,),
        in_specs=[
            pl.BlockSpec((RB, W), lambda i: (i, 0)),
            pl.BlockSpec((RB, W), lambda i: (i + NRB, 0)),
            pl.BlockSpec((RB, W), lambda i: (i, 0)),
            _DSPEC0, _DSPEC1,
            pl.BlockSpec((W, 2048), lambda i: (0, 0)),
            pl.BlockSpec((1, 2048), lambda i: (0, 0)),
        ],
        out_specs=pl.BlockSpec((16, RB, W), lambda i: (0, i, 0)),
        out_shape=jax.ShapeDtypeStruct((16, N, W), _f32),
    )(parts, parts, g0, dparts, dparts, w1, b1)


def _k2a(agg1, g1, dparts, w2, b2):
    def body(agg_ref, g_ref, d0, d1, w_ref, b_ref, o_ref):
        a = _ablock(d0, d1)
        acc = jnp.zeros((RB, 2048), _f32)
        for k in range(16):
            m = a * (agg_ref[k] + g_ref[k])
            acc = acc + _dot(m, w_ref[k * W:(k + 1) * W, :])
        o_ref[...] = jnp.tanh(acc + b_ref[...])

    return pl.pallas_call(
        body,
        grid=(NRB,),
        in_specs=[
            pl.BlockSpec((16, RB, W), lambda i: (0, i, 0)),
            pl.BlockSpec((16, RB, W), lambda i: (0, i, 0)),
            _DSPEC0, _DSPEC1,
            pl.BlockSpec((2048, 2048), lambda i: (0, 0)),
            pl.BlockSpec((1, 2048), lambda i: (0, 0)),
        ],
        out_specs=pl.BlockSpec((RB, 2048), lambda i: (i, 0)),
        out_shape=jax.ShapeDtypeStruct((N, 2048), _f32),
    )(agg1, g1, dparts, dparts, w2, b2)


def _k2b(h2, dparts, w3):
    def body(h_ref, d0, d1, w_ref, o_ref):
        a = _ablock(d0, d1)
        v = a * _dot(h_ref[...], w_ref[...])
        for j in range(8):
            o_ref[j] = v[:, j * W:(j + 1) * W]

    return pl.pallas_call(
        body,
        grid=(NRB,),
        in_specs=[
            pl.BlockSpec((RB, 2048), lambda i: (i, 0)),
            _DSPEC0, _DSPEC1,
            pl.BlockSpec((2048, 1024), lambda i: (0, 0)),
        ],
        out_specs=pl.BlockSpec((8, RB, W), lambda i: (0, i, 0)),
        out_shape=jax.ShapeDtypeStruct((8, N, W), _f32),
    )(h2, dparts, dparts, w3)


def _k3(agg2, g2, dparts, b3, w4, b4):
    def body(agg_ref, g_ref, d0, d1, b3_ref, w_ref, b4_ref, o_ref):
        a = _ablock(d0, d1)
        acc = jnp.zeros((RB, 768), _f32)
        for k in range(8):
            t = jnp.tanh(a * (agg_ref[k] + g_ref[k])
                         + b3_ref[:, k * W:(k + 1) * W])
            acc = acc + _dot(t, w_ref[k * W:(k + 1) * W, :])
        o_ref[...] = acc + b4_ref[...]

    return pl.pallas_call(
        body,
        grid=(NRB,),
        in_specs=[
            pl.BlockSpec((8, RB, W), lambda i: (0, i, 0)),
            pl.BlockSpec((8, RB, W), lambda i: (0, i, 0)),
            _DSPEC0, _DSPEC1,
            pl.BlockSpec((1, 1024), lambda i: (0, 0)),
            pl.BlockSpec((1024, 768), lambda i: (0, 0)),
            pl.BlockSpec((1, 768), lambda i: (0, 0)),
        ],
        out_specs=pl.BlockSpec((RB, 768), lambda i: (i, 0)),
        out_shape=jax.ShapeDtypeStruct((N, 768), _f32),
    )(agg2, g2, dparts, dparts, b3, w4, b4)


def kernel(x, edge_index, W1, b1, W2, b2, W3, b3, W4, b4):
    ei = edge_index.astype(jnp.int32)
    src, dst = ei[0], ei[1]

    srcx16 = _chunked_src(src, 16)

    dparts = _sc_degree(dst)
    g0 = _k0(x, dparts)
    p = _sc_aggregate(src, dst, g0, 1, True)
    g1 = _k1(p, g0, dparts, W1, b1.reshape(1, -1))
    agg1 = _sc_aggregate(srcx16, dst, g1.reshape(16 * N, W), 16, False)
    h2 = _k2a(agg1.reshape(16, N, W), g1, dparts, W2, b2.reshape(1, -1))
    g2 = _k2b(h2, dparts, W3)
    agg2 = _sc_aggregate(srcx16[:8 * E], dst, g2.reshape(8 * N, W), 8, False)
    out = _k3(agg2.reshape(8, N, W), g2, dparts,
              b3.reshape(1, -1), W4, b4.reshape(1, -1))
    return out
